# baseline (speedup 1.0000x reference)
import jax
import jax.numpy as jnp
from jax.experimental import pallas as pl
from jax.experimental.pallas import tpu as pltpu

N = 512
D_HID = 128
N_HEADS = 3
N_LAYERS = 3
BJ = 32
NEG = -1e30
BN_EPS = 1e-5
HIGH = jax.lax.Precision.HIGHEST


def _gat_kernel(x_ref, adj_ref, *rest):
    n_w = N_HEADS * (N_LAYERS * 6 + (N_LAYERS - 1) * 2) + 3 * 2
    w = rest[:n_w]
    o_ref = rest[n_w]
    (lt_ref, mask_ref, xl_ref, xlp_ref, xrp_ref, ar_ref, h_ref,
     hcat_ref) = rest[n_w + 1:]

    adjt = adj_ref[:].T
    jj = jax.lax.broadcasted_iota(jnp.int32, (N, N), 0)
    ii = jax.lax.broadcasted_iota(jnp.int32, (N, N), 1)
    diag = jj == ii
    valid = jnp.logical_or(jnp.logical_and(adjt != 0, jnp.logical_not(diag)), diag)
    mask_ref[:] = jnp.where(valid, 0.0, NEG)

    def conv(h_in, Wl_ref, bl_ref, Wr_ref, br_ref, att_ref, bias_ref):
        xl = jnp.dot(h_in, Wl_ref[:], preferred_element_type=jnp.float32, precision=HIGH) + bl_ref[:]
        xr = jnp.dot(h_in, Wr_ref[:], preferred_element_type=jnp.float32, precision=HIGH) + br_ref[:]
        att = att_ref[:][0]
        xl_ref[:] = xl
        xrp_ref[:] = xr

        def body(jb, _):
            sl = pl.ds(jb * BJ, BJ)
            t = xl[None, :, :] + xrp_ref[sl, :][:, None, :]
            t = jnp.maximum(t, 0.2 * t)
            lg = jnp.sum(t * att[None, None, :], axis=-1)
            lt_ref[sl, :] = lg + mask_ref[sl, :]
            return 0

        jax.lax.fori_loop(0, N // BJ, body, 0)
        ltm = lt_ref[:]
        m = jnp.max(ltm, axis=1, keepdims=True)
        p = jnp.exp(ltm - m)
        denom = jnp.sum(p, axis=1, keepdims=True)
        alpha = p / (denom + 1e-16)
        return jnp.dot(alpha, xl_ref[:], preferred_element_type=jnp.float32, precision=HIGH) + bias_ref[:]

    per_head = N_LAYERS * 6 + (N_LAYERS - 1) * 2
    bn_scale = 1.0 / jnp.sqrt(1.0 + BN_EPS)
    for h in range(N_HEADS):
        base = h * per_head
        for l in range(N_LAYERS):
            cb = base + l * 6
            h_in = x_ref[:] if l == 0 else h_ref[:]
            hcur = conv(h_in, w[cb], w[cb + 1], w[cb + 2], w[cb + 3], w[cb + 4], w[cb + 5])
            if l < N_LAYERS - 1:
                bb = base + N_LAYERS * 6 + l * 2
                hcur = w[bb][:] * hcur * bn_scale + w[bb + 1][:]
                h_ref[:] = jnp.maximum(hcur, 0.0)
            else:
                hcat_ref[:, h * D_HID:(h + 1) * D_HID] = hcur

    out = hcat_ref[:]
    lbase = N_HEADS * per_head
    for i in range(3):
        out = jnp.dot(out, w[lbase + 2 * i][:], preferred_element_type=jnp.float32, precision=HIGH) + w[lbase + 2 * i + 1][:]
        if i < 2:
            out = jnp.maximum(out, 0.0)
    o_ref[:] = out


def kernel(x, adj, batch, params):
    del batch
    flat = []
    for hp in params['heads']:
        for c in hp['convs']:
            flat += [c['Wl'], c['bl'].reshape(1, -1), c['Wr'], c['br'].reshape(1, -1),
                     c['att'].reshape(1, -1), c['bias'].reshape(1, -1)]
        for bn in hp['bns']:
            flat += [bn['gamma'].reshape(1, -1), bn['beta'].reshape(1, -1)]
    for lin in params['linears']:
        flat += [lin['W'], lin['b'].reshape(1, -1)]

    return pl.pallas_call(
        _gat_kernel,
        out_shape=jax.ShapeDtypeStruct((N, params['linears'][-1]['W'].shape[1]), jnp.float32),
        scratch_shapes=[
            pltpu.VMEM((N, N), jnp.float32),
            pltpu.VMEM((N, N), jnp.float32),
            pltpu.VMEM((N, D_HID), jnp.float32),
            pltpu.VMEM((N, D_HID), jnp.float32),
            pltpu.VMEM((N, D_HID), jnp.float32),
            pltpu.VMEM((N, 1), jnp.float32),
            pltpu.VMEM((N, D_HID), jnp.float32),
            pltpu.VMEM((N, N_HEADS * D_HID), jnp.float32),
        ],
    )(x, adj, *flat)


# speedup vs baseline: 2.7419x; 2.7419x over previous
import jax
import jax.numpy as jnp
from jax.experimental import pallas as pl
from jax.experimental.pallas import tpu as pltpu

N = 512
D_HID = 128
N_HEADS = 3
N_LAYERS = 3
BJ = 32
NEG = -1e30
BN_EPS = 1e-5


def _gat_kernel(x_ref, adj_ref, *rest):
    n_w = N_HEADS * (N_LAYERS * 6 + (N_LAYERS - 1) * 2) + 3 * 2
    w = rest[:n_w]
    o_ref = rest[n_w]
    lt_ref = rest[n_w + 1]
    xr_ref = rest[n_w + 2]

    adjt = adj_ref[:].T
    jj = jax.lax.broadcasted_iota(jnp.int32, (N, N), 0)
    ii = jax.lax.broadcasted_iota(jnp.int32, (N, N), 1)
    diag = jj == ii
    valid = jnp.logical_or(jnp.logical_and(adjt != 0, jnp.logical_not(diag)), diag)
    mneg = jnp.where(valid, 0.0, NEG)

    def conv(h_in, Wl_ref, bl_ref, Wr_ref, br_ref, att_ref, bias_ref):
        xl = jnp.dot(h_in, Wl_ref[:], preferred_element_type=jnp.float32, precision=jax.lax.Precision.HIGHEST) + bl_ref[:]
        xr_ref[:] = jnp.dot(h_in, Wr_ref[:], preferred_element_type=jnp.float32, precision=jax.lax.Precision.HIGHEST) + br_ref[:]
        att = att_ref[:][0]

        def body(jb, _):
            xr_blk = xr_ref[pl.ds(jb * BJ, BJ), :]
            t = xl[None, :, :] + xr_blk[:, None, :]
            t = jnp.where(t > 0, t, 0.2 * t)
            lg = jnp.sum(t * att[None, None, :], axis=-1)
            lt_ref[pl.ds(jb * BJ, BJ), :] = lg
            return 0

        jax.lax.fori_loop(0, N // BJ, body, 0)
        ltm = lt_ref[:] + mneg
        m = jnp.max(ltm, axis=1, keepdims=True)
        p = jnp.exp(ltm - m)
        denom = jnp.sum(p, axis=1, keepdims=True)
        alpha = p / (denom + 1e-16)
        return jnp.dot(alpha, xl, preferred_element_type=jnp.float32, precision=jax.lax.Precision.HIGHEST) + bias_ref[:]

    head_outs = []
    per_head = N_LAYERS * 6 + (N_LAYERS - 1) * 2
    bn_scale = 1.0 / jnp.sqrt(1.0 + BN_EPS)
    for h in range(N_HEADS):
        base = h * per_head
        hcur = x_ref[:]
        for l in range(N_LAYERS):
            cb = base + l * 6
            hcur = conv(hcur, w[cb], w[cb + 1], w[cb + 2], w[cb + 3], w[cb + 4], w[cb + 5])
            if l < N_LAYERS - 1:
                bb = base + N_LAYERS * 6 + l * 2
                hcur = w[bb][:] * hcur * bn_scale + w[bb + 1][:]
                hcur = jnp.maximum(hcur, 0.0)
        head_outs.append(hcur)

    out = jnp.concatenate(head_outs, axis=1)
    lbase = N_HEADS * per_head
    for i in range(3):
        out = jnp.dot(out, w[lbase + 2 * i][:], preferred_element_type=jnp.float32, precision=jax.lax.Precision.HIGHEST) + w[lbase + 2 * i + 1][:]
        if i < 2:
            out = jnp.maximum(out, 0.0)
    o_ref[:] = out


def kernel(x, adj, batch, params):
    del batch
    flat = []
    for hp in params['heads']:
        for c in hp['convs']:
            flat += [c['Wl'], c['bl'].reshape(1, -1), c['Wr'], c['br'].reshape(1, -1),
                     c['att'].reshape(1, -1), c['bias'].reshape(1, -1)]
        for bn in hp['bns']:
            flat += [bn['gamma'].reshape(1, -1), bn['beta'].reshape(1, -1)]
    for lin in params['linears']:
        flat += [lin['W'], lin['b'].reshape(1, -1)]

    return pl.pallas_call(
        _gat_kernel,
        out_shape=jax.ShapeDtypeStruct((N, params['linears'][-1]['W'].shape[1]), jnp.float32),
        scratch_shapes=[pltpu.VMEM((N, N), jnp.float32),
                        pltpu.VMEM((N, D_HID), jnp.float32)],
    )(x, adj, *flat)


# speedup vs baseline: 2.7437x; 1.0007x over previous
import jax
import jax.numpy as jnp
from jax.experimental import pallas as pl
from jax.experimental.pallas import tpu as pltpu

N = 512
D_HID = 128
N_HEADS = 3
N_LAYERS = 3
BJ = 32
NEG = -1e30
BN_EPS = 1e-5


def _gat_kernel(x_ref, adj_ref, *rest):
    n_w = N_HEADS * (N_LAYERS * 6 + (N_LAYERS - 1) * 2) + 3 * 2
    w = rest[:n_w]
    o_ref = rest[n_w]
    lt_ref = rest[n_w + 1]
    xr_ref = rest[n_w + 2]

    adjt = adj_ref[:].T
    jj = jax.lax.broadcasted_iota(jnp.int32, (N, N), 0)
    ii = jax.lax.broadcasted_iota(jnp.int32, (N, N), 1)
    diag = jj == ii
    valid = jnp.logical_or(jnp.logical_and(adjt != 0, jnp.logical_not(diag)), diag)
    mneg = jnp.where(valid, 0.0, NEG)

    def conv(h_in, Wl_ref, bl_ref, Wr_ref, br_ref, att_ref, bias_ref):
        xl = jnp.dot(h_in, Wl_ref[:], preferred_element_type=jnp.float32, precision=jax.lax.Precision.HIGHEST) + bl_ref[:]
        xr_ref[:] = jnp.dot(h_in, Wr_ref[:], preferred_element_type=jnp.float32, precision=jax.lax.Precision.HIGHEST) + br_ref[:]
        att = att_ref[:][0]

        def body(jb, _):
            xr_blk = xr_ref[pl.ds(jb * BJ, BJ), :]
            t = xl[None, :, :] + xr_blk[:, None, :]
            t = jnp.maximum(t, 0.2 * t)
            lg = jnp.sum(t * att[None, None, :], axis=-1)
            lt_ref[pl.ds(jb * BJ, BJ), :] = lg
            return 0

        jax.lax.fori_loop(0, N // BJ, body, 0)
        ltm = lt_ref[:] + mneg
        m = jnp.max(ltm, axis=1, keepdims=True)
        p = jnp.exp(ltm - m)
        denom = jnp.sum(p, axis=1, keepdims=True)
        alpha = p / (denom + 1e-16)
        return jnp.dot(alpha, xl, preferred_element_type=jnp.float32, precision=jax.lax.Precision.HIGHEST) + bias_ref[:]

    head_outs = []
    per_head = N_LAYERS * 6 + (N_LAYERS - 1) * 2
    bn_scale = 1.0 / jnp.sqrt(1.0 + BN_EPS)
    for h in range(N_HEADS):
        base = h * per_head
        hcur = x_ref[:]
        for l in range(N_LAYERS):
            cb = base + l * 6
            hcur = conv(hcur, w[cb], w[cb + 1], w[cb + 2], w[cb + 3], w[cb + 4], w[cb + 5])
            if l < N_LAYERS - 1:
                bb = base + N_LAYERS * 6 + l * 2
                hcur = w[bb][:] * hcur * bn_scale + w[bb + 1][:]
                hcur = jnp.maximum(hcur, 0.0)
        head_outs.append(hcur)

    out = jnp.concatenate(head_outs, axis=1)
    lbase = N_HEADS * per_head
    for i in range(3):
        out = jnp.dot(out, w[lbase + 2 * i][:], preferred_element_type=jnp.float32, precision=jax.lax.Precision.HIGHEST) + w[lbase + 2 * i + 1][:]
        if i < 2:
            out = jnp.maximum(out, 0.0)
    o_ref[:] = out


def kernel(x, adj, batch, params):
    del batch
    flat = []
    for hp in params['heads']:
        for c in hp['convs']:
            flat += [c['Wl'], c['bl'].reshape(1, -1), c['Wr'], c['br'].reshape(1, -1),
                     c['att'].reshape(1, -1), c['bias'].reshape(1, -1)]
        for bn in hp['bns']:
            flat += [bn['gamma'].reshape(1, -1), bn['beta'].reshape(1, -1)]
    for lin in params['linears']:
        flat += [lin['W'], lin['b'].reshape(1, -1)]

    return pl.pallas_call(
        _gat_kernel,
        out_shape=jax.ShapeDtypeStruct((N, params['linears'][-1]['W'].shape[1]), jnp.float32),
        scratch_shapes=[pltpu.VMEM((N, N), jnp.float32),
                        pltpu.VMEM((N, D_HID), jnp.float32)],
    )(x, adj, *flat)


# speedup vs baseline: 2.7822x; 1.0141x over previous
import jax
import jax.numpy as jnp
from jax.experimental import pallas as pl
from jax.experimental.pallas import tpu as pltpu

N = 512
D_HID = 128
N_HEADS = 3
N_LAYERS = 3
BJ = 64
NEG = -1e30
BN_EPS = 1e-5


def _gat_kernel(x_ref, adj_ref, *rest):
    n_w = N_HEADS * (N_LAYERS * 6 + (N_LAYERS - 1) * 2) + 3 * 2
    w = rest[:n_w]
    o_ref = rest[n_w]
    lt_ref = rest[n_w + 1]
    xr_ref = rest[n_w + 2]

    adjt = adj_ref[:].T
    jj = jax.lax.broadcasted_iota(jnp.int32, (N, N), 0)
    ii = jax.lax.broadcasted_iota(jnp.int32, (N, N), 1)
    diag = jj == ii
    valid = jnp.logical_or(jnp.logical_and(adjt != 0, jnp.logical_not(diag)), diag)
    mneg = jnp.where(valid, 0.0, NEG)

    def conv(h_in, Wl_ref, bl_ref, Wr_ref, br_ref, att_ref, bias_ref):
        xl = jnp.dot(h_in, Wl_ref[:], preferred_element_type=jnp.float32, precision=jax.lax.Precision.HIGHEST) + bl_ref[:]
        xr_ref[:] = jnp.dot(h_in, Wr_ref[:], preferred_element_type=jnp.float32, precision=jax.lax.Precision.HIGHEST) + br_ref[:]
        att = att_ref[:][0]

        def body(jb, _):
            xr_blk = xr_ref[pl.ds(jb * BJ, BJ), :]
            t = xl[None, :, :] + xr_blk[:, None, :]
            t = jnp.maximum(t, 0.2 * t)
            lg = jnp.sum(t * att[None, None, :], axis=-1)
            lt_ref[pl.ds(jb * BJ, BJ), :] = lg
            return 0

        jax.lax.fori_loop(0, N // BJ, body, 0)
        ltm = lt_ref[:] + mneg
        m = jnp.max(ltm, axis=1, keepdims=True)
        p = jnp.exp(ltm - m)
        denom = jnp.sum(p, axis=1, keepdims=True)
        alpha = p / (denom + 1e-16)
        return jnp.dot(alpha, xl, preferred_element_type=jnp.float32, precision=jax.lax.Precision.HIGHEST) + bias_ref[:]

    head_outs = []
    per_head = N_LAYERS * 6 + (N_LAYERS - 1) * 2
    bn_scale = 1.0 / jnp.sqrt(1.0 + BN_EPS)
    for h in range(N_HEADS):
        base = h * per_head
        hcur = x_ref[:]
        for l in range(N_LAYERS):
            cb = base + l * 6
            hcur = conv(hcur, w[cb], w[cb + 1], w[cb + 2], w[cb + 3], w[cb + 4], w[cb + 5])
            if l < N_LAYERS - 1:
                bb = base + N_LAYERS * 6 + l * 2
                hcur = w[bb][:] * hcur * bn_scale + w[bb + 1][:]
                hcur = jnp.maximum(hcur, 0.0)
        head_outs.append(hcur)

    out = jnp.concatenate(head_outs, axis=1)
    lbase = N_HEADS * per_head
    for i in range(3):
        out = jnp.dot(out, w[lbase + 2 * i][:], preferred_element_type=jnp.float32, precision=jax.lax.Precision.HIGHEST) + w[lbase + 2 * i + 1][:]
        if i < 2:
            out = jnp.maximum(out, 0.0)
    o_ref[:] = out


def kernel(x, adj, batch, params):
    del batch
    flat = []
    for hp in params['heads']:
        for c in hp['convs']:
            flat += [c['Wl'], c['bl'].reshape(1, -1), c['Wr'], c['br'].reshape(1, -1),
                     c['att'].reshape(1, -1), c['bias'].reshape(1, -1)]
        for bn in hp['bns']:
            flat += [bn['gamma'].reshape(1, -1), bn['beta'].reshape(1, -1)]
    for lin in params['linears']:
        flat += [lin['W'], lin['b'].reshape(1, -1)]

    return pl.pallas_call(
        _gat_kernel,
        out_shape=jax.ShapeDtypeStruct((N, params['linears'][-1]['W'].shape[1]), jnp.float32),
        scratch_shapes=[pltpu.VMEM((N, N), jnp.float32),
                        pltpu.VMEM((N, D_HID), jnp.float32)],
    )(x, adj, *flat)


# speedup vs baseline: 2.8027x; 1.0073x over previous
import jax
import jax.numpy as jnp
from jax.experimental import pallas as pl
from jax.experimental.pallas import tpu as pltpu

N = 512
D_HID = 128
N_HEADS = 3
N_LAYERS = 3
BJ = 128
NEG = -1e30
BN_EPS = 1e-5


def _gat_kernel(x_ref, adj_ref, *rest):
    n_w = N_HEADS * (N_LAYERS * 6 + (N_LAYERS - 1) * 2) + 3 * 2
    w = rest[:n_w]
    o_ref = rest[n_w]
    lt_ref = rest[n_w + 1]
    xr_ref = rest[n_w + 2]

    adjt = adj_ref[:].T
    jj = jax.lax.broadcasted_iota(jnp.int32, (N, N), 0)
    ii = jax.lax.broadcasted_iota(jnp.int32, (N, N), 1)
    diag = jj == ii
    valid = jnp.logical_or(jnp.logical_and(adjt != 0, jnp.logical_not(diag)), diag)
    mneg = jnp.where(valid, 0.0, NEG)

    def conv(h_in, Wl_ref, bl_ref, Wr_ref, br_ref, att_ref, bias_ref):
        xl = jnp.dot(h_in, Wl_ref[:], preferred_element_type=jnp.float32, precision=jax.lax.Precision.HIGHEST) + bl_ref[:]
        xr_ref[:] = jnp.dot(h_in, Wr_ref[:], preferred_element_type=jnp.float32, precision=jax.lax.Precision.HIGHEST) + br_ref[:]
        att = att_ref[:][0]

        def body(jb, _):
            xr_blk = xr_ref[pl.ds(jb * BJ, BJ), :]
            t = xl[None, :, :] + xr_blk[:, None, :]
            t = jnp.maximum(t, 0.2 * t)
            lg = jnp.sum(t * att[None, None, :], axis=-1)
            lt_ref[pl.ds(jb * BJ, BJ), :] = lg
            return 0

        jax.lax.fori_loop(0, N // BJ, body, 0)
        ltm = lt_ref[:] + mneg
        m = jnp.max(ltm, axis=1, keepdims=True)
        p = jnp.exp(ltm - m)
        denom = jnp.sum(p, axis=1, keepdims=True)
        alpha = p / (denom + 1e-16)
        return jnp.dot(alpha, xl, preferred_element_type=jnp.float32, precision=jax.lax.Precision.HIGHEST) + bias_ref[:]

    head_outs = []
    per_head = N_LAYERS * 6 + (N_LAYERS - 1) * 2
    bn_scale = 1.0 / jnp.sqrt(1.0 + BN_EPS)
    for h in range(N_HEADS):
        base = h * per_head
        hcur = x_ref[:]
        for l in range(N_LAYERS):
            cb = base + l * 6
            hcur = conv(hcur, w[cb], w[cb + 1], w[cb + 2], w[cb + 3], w[cb + 4], w[cb + 5])
            if l < N_LAYERS - 1:
                bb = base + N_LAYERS * 6 + l * 2
                hcur = w[bb][:] * hcur * bn_scale + w[bb + 1][:]
                hcur = jnp.maximum(hcur, 0.0)
        head_outs.append(hcur)

    out = jnp.concatenate(head_outs, axis=1)
    lbase = N_HEADS * per_head
    for i in range(3):
        out = jnp.dot(out, w[lbase + 2 * i][:], preferred_element_type=jnp.float32, precision=jax.lax.Precision.HIGHEST) + w[lbase + 2 * i + 1][:]
        if i < 2:
            out = jnp.maximum(out, 0.0)
    o_ref[:] = out


def kernel(x, adj, batch, params):
    del batch
    flat = []
    for hp in params['heads']:
        for c in hp['convs']:
            flat += [c['Wl'], c['bl'].reshape(1, -1), c['Wr'], c['br'].reshape(1, -1),
                     c['att'].reshape(1, -1), c['bias'].reshape(1, -1)]
        for bn in hp['bns']:
            flat += [bn['gamma'].reshape(1, -1), bn['beta'].reshape(1, -1)]
    for lin in params['linears']:
        flat += [lin['W'], lin['b'].reshape(1, -1)]

    return pl.pallas_call(
        _gat_kernel,
        out_shape=jax.ShapeDtypeStruct((N, params['linears'][-1]['W'].shape[1]), jnp.float32),
        scratch_shapes=[pltpu.VMEM((N, N), jnp.float32),
                        pltpu.VMEM((N, D_HID), jnp.float32)],
    )(x, adj, *flat)


# speedup vs baseline: 2.9792x; 1.0630x over previous
import jax
import jax.numpy as jnp
from jax.experimental import pallas as pl
from jax.experimental.pallas import tpu as pltpu

N = 512
D_HID = 128
N_HEADS = 3
N_LAYERS = 3
BJ = 128
NEG = -1e30
BN_EPS = 1e-5


def _gat_kernel(x_ref, adj_ref, *rest):
    n_w = N_HEADS * (N_LAYERS * 6 + (N_LAYERS - 1) * 2) + 3 * 2
    w = rest[:n_w]
    o_ref = rest[n_w]
    lt_ref = rest[n_w + 1]
    xr_ref = rest[n_w + 2]

    adjt = adj_ref[:].T
    jj = jax.lax.broadcasted_iota(jnp.int32, (N, N), 0)
    ii = jax.lax.broadcasted_iota(jnp.int32, (N, N), 1)
    diag = jj == ii
    valid = jnp.logical_or(jnp.logical_and(adjt != 0, jnp.logical_not(diag)), diag)
    mneg = jnp.where(valid, 0.0, NEG)

    def conv(h_in, Wl_ref, bl_ref, Wr_ref, br_ref, att_ref, bias_ref):
        xl = jnp.dot(h_in, Wl_ref[:], preferred_element_type=jnp.float32, precision=jax.lax.Precision.HIGHEST) + bl_ref[:]
        xr_ref[:] = jnp.dot(h_in, Wr_ref[:], preferred_element_type=jnp.float32, precision=jax.lax.Precision.HIGHEST) + br_ref[:]
        att = att_ref[:][0]

        def body(jb, _):
            xr_blk = xr_ref[pl.ds(jb * BJ, BJ), :]
            t = xl[None, :, :] + xr_blk[:, None, :]
            t = jnp.maximum(t, 0.2 * t)
            lg = jnp.sum(t * att[None, None, :], axis=-1)
            lt_ref[pl.ds(jb * BJ, BJ), :] = lg
            return 0

        jax.lax.fori_loop(0, N // BJ, body, 0)
        ltm = lt_ref[:] + mneg
        m = jnp.max(ltm, axis=1, keepdims=True)
        p = jnp.exp(ltm - m)
        denom = jnp.sum(p, axis=1, keepdims=True)
        alpha = p / (denom + 1e-16)
        return jnp.dot(alpha, xl, preferred_element_type=jnp.float32) + bias_ref[:]

    head_outs = []
    per_head = N_LAYERS * 6 + (N_LAYERS - 1) * 2
    bn_scale = 1.0 / jnp.sqrt(1.0 + BN_EPS)
    for h in range(N_HEADS):
        base = h * per_head
        hcur = x_ref[:]
        for l in range(N_LAYERS):
            cb = base + l * 6
            hcur = conv(hcur, w[cb], w[cb + 1], w[cb + 2], w[cb + 3], w[cb + 4], w[cb + 5])
            if l < N_LAYERS - 1:
                bb = base + N_LAYERS * 6 + l * 2
                hcur = w[bb][:] * hcur * bn_scale + w[bb + 1][:]
                hcur = jnp.maximum(hcur, 0.0)
        head_outs.append(hcur)

    out = jnp.concatenate(head_outs, axis=1)
    lbase = N_HEADS * per_head
    for i in range(3):
        out = jnp.dot(out, w[lbase + 2 * i][:], preferred_element_type=jnp.float32) + w[lbase + 2 * i + 1][:]
        if i < 2:
            out = jnp.maximum(out, 0.0)
    o_ref[:] = out


def kernel(x, adj, batch, params):
    del batch
    flat = []
    for hp in params['heads']:
        for c in hp['convs']:
            flat += [c['Wl'], c['bl'].reshape(1, -1), c['Wr'], c['br'].reshape(1, -1),
                     c['att'].reshape(1, -1), c['bias'].reshape(1, -1)]
        for bn in hp['bns']:
            flat += [bn['gamma'].reshape(1, -1), bn['beta'].reshape(1, -1)]
    for lin in params['linears']:
        flat += [lin['W'], lin['b'].reshape(1, -1)]

    return pl.pallas_call(
        _gat_kernel,
        out_shape=jax.ShapeDtypeStruct((N, params['linears'][-1]['W'].shape[1]), jnp.float32),
        scratch_shapes=[pltpu.VMEM((N, N), jnp.float32),
                        pltpu.VMEM((N, D_HID), jnp.float32)],
    )(x, adj, *flat)
